# baseline (speedup 1.0000x reference)
import jax
import jax.numpy as jnp
from jax import lax
from jax.experimental import pallas as pl
from jax.experimental.pallas import tpu as pltpu
from jax.experimental.pallas import tpu_sc as plsc

N_VOCAB = 1000000
K = 16
ROWW = 32
BATCH = 16384
FIELDS = 26

NC = 2
NS = 16
NW = NC * NS
SAMPLES_PER_W = BATCH // NW
CHUNK = 64
N_CHUNKS = SAMPLES_PER_W // CHUNK

TCOLS = 1024
N_TCHUNKS = N_VOCAB // TCOLS
TAIL = N_VOCAB - N_TCHUNKS * TCOLS


def _repack_body(embt_hbm, bias_hbm, tail_hbm, out_hbm,
                 in0, in1, bb0, bb1, ou0, ou1,
                 is0, is1, ibs0, ibs1, os0, os1):
    wid = lax.axis_index("s") * NC + lax.axis_index("c")
    iota32 = lax.iota(jnp.int32, 16) * ROWW
    ins, bbs, outs = (in0, in1), (bb0, bb1), (ou0, ou1)
    iss, ibss, oss = (is0, is1), (ibs0, ibs1), (os0, os1)

    n_mine = (N_TCHUNKS - wid + NW - 1) // NW

    def col_off(j):
        return pl.multiple_of((wid + j * NW) * TCOLS, TCOLS)

    def fire_in(j, b):
        co = col_off(j)
        pltpu.async_copy(embt_hbm.at[:, pl.ds(co, TCOLS)], ins[b], iss[b])
        pltpu.async_copy(bias_hbm.at[pl.ds(co, TCOLS)], bbs[b], ibss[b])

    for b in range(2):
        fire_in(b, b)

    def half_body(i, _):
        for b in range(2):
            j = 2 * i + b

            @pl.when(j < n_mine)
            def _(b=b, j=j):
                co = col_off(j)
                pltpu.make_async_copy(
                    embt_hbm.at[:, pl.ds(co, TCOLS)], ins[b], iss[b]).wait()
                pltpu.make_async_copy(
                    bias_hbm.at[pl.ds(co, TCOLS)], bbs[b], ibss[b]).wait()

                @pl.when(j >= 2)
                def _():
                    pltpu.make_async_copy(
                        outs[b], out_hbm.at[pl.ds(0, TCOLS * ROWW)],
                        oss[b]).wait()

                def grp(g, _):
                    c = g * 16
                    cbase = c * ROWW
                    norm = jnp.zeros((16,), jnp.float32)
                    for k in range(K):
                        vec = ins[b][k, pl.ds(c, 16)]
                        norm = norm + vec * vec
                        plsc.store_scatter(outs[b], [iota32 + (cbase + k)],
                                           vec)
                    combo = bbs[b][pl.ds(c, 16)] - 0.5 * norm
                    plsc.store_scatter(outs[b], [iota32 + (cbase + K)], combo)
                    return 0

                lax.fori_loop(0, TCOLS // 16, grp, 0)
                pltpu.async_copy(
                    outs[b], out_hbm.at[pl.ds(co * ROWW, TCOLS * ROWW)],
                    oss[b])

                @pl.when(j + 2 < n_mine)
                def _():
                    fire_in(j + 2, b)
        return 0

    lax.fori_loop(0, (N_TCHUNKS // NW + 2) // 2, half_body, 0)

    for b in range(2):
        pltpu.make_async_copy(
            outs[b], out_hbm.at[pl.ds(0, TCOLS * ROWW)], oss[b]).wait()

    @pl.when(wid == NW - 1)
    def _():
        pltpu.sync_copy(tail_hbm, outs[0].at[pl.ds(0, TAIL * ROWW)])
        pltpu.sync_copy(outs[0].at[pl.ds(0, TAIL * ROWW)],
                        out_hbm.at[pl.ds(N_TCHUNKS * TCOLS * ROWW,
                                         TAIL * ROWW)])


def _fm_body(x_hbm, tab_hbm, w0_hbm, out_hbm,
             idx0, idx1, emb0, emb1, ou0, ou1, w0_v,
             es0, es1, os0, os1):
    wid = lax.axis_index("s") * NC + lax.axis_index("c")

    pltpu.sync_copy(w0_hbm, w0_v)
    w0vec = w0_v[...]

    idxs, embs = (idx0, idx1), (emb0, emb1)
    outs, ess, oss = (ou0, ou1), (es0, es1), (os0, os1)

    iota26 = lax.iota(jnp.int32, 16) * FIELDS
    combo_col = jnp.full((16,), K, jnp.int32)
    zerof = jnp.zeros((16,), jnp.float32)
    CF = CHUNK * FIELDS

    def fire(c, b):
        off = (wid * N_CHUNKS + c) * CF
        pltpu.sync_copy(x_hbm.at[pl.ds(off, CF)], idxs[b])
        pltpu.async_copy(tab_hbm.at[idxs[b]], embs[b], ess[b])

    fire(0, 0)
    for c in range(N_CHUNKS):
        b = c % 2
        if c + 1 < N_CHUNKS:
            fire(c + 1, 1 - b)
        pltpu.make_async_copy(tab_hbm.at[idxs[b]], embs[b], ess[b]).wait()
        if c >= 2:
            pltpu.make_async_copy(
                outs[b], out_hbm.at[pl.ds(0, CHUNK)], oss[b]).wait()

        def group_body(g, _, b=b):
            rbase = iota26 + g * (16 * FIELDS)
            rows = [rbase + f for f in range(FIELDS)]

            def k_body(k, acc):
                cols = jnp.full((16,), k, jnp.int32)
                s = [zerof, zerof, zerof, zerof]
                for f in range(FIELDS):
                    v = plsc.load_gather(embs[b], [rows[f], cols])
                    s[f % 4] = s[f % 4] + v
                st = (s[0] + s[1]) + (s[2] + s[3])
                return acc + st * st

            pair = lax.fori_loop(0, K, k_body, zerof)
            ba = [zerof, zerof]
            for f in range(FIELDS):
                ba[f % 2] = ba[f % 2] + plsc.load_gather(
                    embs[b], [rows[f], combo_col])
            t = w0vec + (ba[0] + ba[1]) + 0.5 * pair
            outs[b][pl.ds(g * 16, 16)] = 5.5 / (1.0 + jnp.exp(-t))
            return 0

        lax.fori_loop(0, CHUNK // 16, group_body, 0)

        out_off = wid * SAMPLES_PER_W + c * CHUNK
        pltpu.async_copy(outs[b], out_hbm.at[pl.ds(out_off, CHUNK)], oss[b])

    for b in range(2):
        pltpu.make_async_copy(
            outs[b], out_hbm.at[pl.ds(0, CHUNK)], oss[b]).wait()


def _sc_mesh():
    return plsc.VectorSubcoreMesh(core_axis_name="c", subcore_axis_name="s")


@jax.jit
def _fm_call(X, emb, bias, w0):
    xflat = X.reshape(BATCH * FIELDS)
    w0b = jnp.broadcast_to(w0, (16,))
    bias_lin = bias.T.reshape(N_VOCAB)
    emb_tail = emb[N_TCHUNKS * TCOLS:, :]
    combo_tail = (bias[N_TCHUNKS * TCOLS:, :]
                  - 0.5 * jnp.sum(emb_tail * emb_tail, axis=1, keepdims=True))
    tail = jnp.concatenate(
        [emb_tail, combo_tail, jnp.zeros((TAIL, ROWW - K - 1), jnp.float32)],
        axis=1).reshape(TAIL * ROWW)

    table = pl.kernel(
        _repack_body,
        out_type=jax.ShapeDtypeStruct((N_VOCAB * ROWW,), jnp.float32),
        mesh=_sc_mesh(),
        scratch_types=[
            pltpu.VMEM((16, TCOLS), jnp.float32),
            pltpu.VMEM((16, TCOLS), jnp.float32),
            pltpu.VMEM((TCOLS,), jnp.float32),
            pltpu.VMEM((TCOLS,), jnp.float32),
            pltpu.VMEM((TCOLS * ROWW,), jnp.float32),
            pltpu.VMEM((TCOLS * ROWW,), jnp.float32),
            pltpu.SemaphoreType.DMA,
            pltpu.SemaphoreType.DMA,
            pltpu.SemaphoreType.DMA,
            pltpu.SemaphoreType.DMA,
            pltpu.SemaphoreType.DMA,
            pltpu.SemaphoreType.DMA,
        ],
        compiler_params=pltpu.CompilerParams(
            needs_layout_passes=False, use_tc_tiling_on_sc=True),
    )(emb.T, bias_lin, tail)

    return pl.kernel(
        _fm_body,
        out_type=jax.ShapeDtypeStruct((BATCH,), jnp.float32),
        mesh=_sc_mesh(),
        scratch_types=[
            pltpu.VMEM((CHUNK * FIELDS,), jnp.int32),
            pltpu.VMEM((CHUNK * FIELDS,), jnp.int32),
            pltpu.VMEM((CHUNK * FIELDS, ROWW), jnp.float32),
            pltpu.VMEM((CHUNK * FIELDS, ROWW), jnp.float32),
            pltpu.VMEM((CHUNK,), jnp.float32),
            pltpu.VMEM((CHUNK,), jnp.float32),
            pltpu.VMEM((16,), jnp.float32),
            pltpu.SemaphoreType.DMA,
            pltpu.SemaphoreType.DMA,
            pltpu.SemaphoreType.DMA,
            pltpu.SemaphoreType.DMA,
        ],
        compiler_params=pltpu.CompilerParams(
            needs_layout_passes=False, use_tc_tiling_on_sc=False),
    )(xflat, table.reshape(N_VOCAB, ROWW), w0b)


def kernel(X, embeddings, bias, w0):
    return _fm_call(X.astype(jnp.int32), embeddings,
                    bias.astype(jnp.float32), w0.astype(jnp.float32))


# speedup vs baseline: 2.1941x; 2.1941x over previous
import jax
import jax.numpy as jnp
from jax import lax
from jax.experimental import pallas as pl
from jax.experimental.pallas import tpu as pltpu
from jax.experimental.pallas import tpu_sc as plsc

N_VOCAB = 1000000
K = 16
BATCH = 16384
FIELDS = 26

NC = 2
NS = 16
NW = NC * NS
SAMPLES_PER_W = BATCH // NW
CHUNK = 128
N_CHUNKS = SAMPLES_PER_W // CHUNK

TCOLS = 512
N_TCHUNKS = N_VOCAB // TCOLS
TAIL = N_VOCAB - N_TCHUNKS * TCOLS


def _transpose_body(embt_hbm, tail_hbm, out_hbm,
                    in0, in1, ou0, ou1, is0, is1, os0, os1):
    wid = lax.axis_index("s") * NC + lax.axis_index("c")
    iota16k = lax.iota(jnp.int32, 16) * K
    ins, outs, iss, oss = (in0, in1), (ou0, ou1), (is0, is1), (os0, os1)

    n_mine = (N_TCHUNKS - wid + NW - 1) // NW

    def col_off(j):
        return pl.multiple_of((wid + j * NW) * TCOLS, TCOLS)

    for b in range(2):
        pltpu.async_copy(embt_hbm.at[:, pl.ds(col_off(b), TCOLS)],
                         ins[b], iss[b])

    def half_body(i, _):
        for b in range(2):
            j = 2 * i + b

            @pl.when(j < n_mine)
            def _(b=b, j=j):
                co = col_off(j)
                pltpu.make_async_copy(
                    embt_hbm.at[:, pl.ds(co, TCOLS)], ins[b], iss[b]).wait()

                @pl.when(j >= 2)
                def _():
                    pltpu.make_async_copy(
                        outs[b], out_hbm.at[pl.ds(0, TCOLS * K)],
                        oss[b]).wait()

                def grp(g, _):
                    c = g * 16
                    cbase = c * K
                    for k in range(K):
                        vec = ins[b][k, pl.ds(c, 16)]
                        plsc.store_scatter(outs[b], [iota16k + (cbase + k)],
                                           vec)
                    return 0

                lax.fori_loop(0, TCOLS // 16, grp, 0)
                pltpu.async_copy(
                    outs[b], out_hbm.at[pl.ds(co * K, TCOLS * K)], oss[b])

                @pl.when(j + 2 < n_mine)
                def _():
                    co2 = col_off(j + 2)
                    pltpu.async_copy(embt_hbm.at[:, pl.ds(co2, TCOLS)],
                                     ins[b], iss[b])
        return 0

    lax.fori_loop(0, (N_TCHUNKS // NW + 2) // 2, half_body, 0)

    for b in range(2):
        pltpu.make_async_copy(
            outs[b], out_hbm.at[pl.ds(0, TCOLS * K)], oss[b]).wait()

    @pl.when(wid == NW - 1)
    def _():
        pltpu.sync_copy(tail_hbm, out_hbm.at[pl.ds(N_TCHUNKS * TCOLS * K,
                                                   TAIL * K)])


def _fm_body(x_hbm, emb_hbm, bias_hbm, w0_hbm, out_hbm,
             idx0, idx1, emb0, emb1, bia0, bia1, ou0, ou1, w0_v,
             es0, es1, bs0, bs1, os0, os1):
    wid = lax.axis_index("s") * NC + lax.axis_index("c")

    pltpu.sync_copy(w0_hbm, w0_v)
    w0vec = w0_v[...]

    idxs, embs, bias_b = (idx0, idx1), (emb0, emb1), (bia0, bia1)
    outs, ess, bss, oss = (ou0, ou1), (es0, es1), (bs0, bs1), (os0, os1)

    iota26 = lax.iota(jnp.int32, 16) * FIELDS
    zerof = jnp.zeros((16,), jnp.float32)
    CF = CHUNK * FIELDS

    def fire(c, b):
        off = (wid * N_CHUNKS + c) * CF
        pltpu.sync_copy(x_hbm.at[pl.ds(off, CF)], idxs[b])
        pltpu.async_copy(emb_hbm.at[idxs[b]], embs[b], ess[b])
        pltpu.async_copy(bias_hbm.at[idxs[b]], bias_b[b], bss[b])

    fire(0, 0)
    for c in range(N_CHUNKS):
        b = c % 2
        if c + 1 < N_CHUNKS:
            fire(c + 1, 1 - b)
        pltpu.make_async_copy(emb_hbm.at[idxs[b]], embs[b], ess[b]).wait()
        pltpu.make_async_copy(bias_hbm.at[idxs[b]], bias_b[b], bss[b]).wait()
        if c >= 2:
            pltpu.make_async_copy(
                outs[b], out_hbm.at[pl.ds(0, CHUNK)], oss[b]).wait()

        def group_body(g, _, b=b):
            rbase = iota26 + g * (16 * FIELDS)
            rows = [rbase + f for f in range(FIELDS)]

            def k_body(k, acc):
                cols = jnp.full((16,), k, jnp.int32)
                s = [zerof, zerof, zerof, zerof]
                q = [zerof, zerof, zerof, zerof]
                for f in range(FIELDS):
                    v = plsc.load_gather(embs[b], [rows[f], cols])
                    s[f % 4] = s[f % 4] + v
                    q[f % 4] = q[f % 4] + v * v
                st = (s[0] + s[1]) + (s[2] + s[3])
                qt = (q[0] + q[1]) + (q[2] + q[3])
                return acc + (st * st - qt)

            pair = lax.fori_loop(0, K, k_body, zerof)
            ba = [zerof, zerof]
            for f in range(FIELDS):
                ba[f % 2] = ba[f % 2] + plsc.load_gather(bias_b[b], [rows[f]])
            t = w0vec + (ba[0] + ba[1]) + 0.5 * pair
            outs[b][pl.ds(g * 16, 16)] = 5.5 / (1.0 + jnp.exp(-t))
            return 0

        lax.fori_loop(0, CHUNK // 16, group_body, 0)

        out_off = wid * SAMPLES_PER_W + c * CHUNK
        pltpu.async_copy(outs[b], out_hbm.at[pl.ds(out_off, CHUNK)], oss[b])

    for b in range(2):
        pltpu.make_async_copy(
            outs[b], out_hbm.at[pl.ds(0, CHUNK)], oss[b]).wait()


def _sc_mesh():
    return plsc.VectorSubcoreMesh(core_axis_name="c", subcore_axis_name="s")


@jax.jit
def _fm_call(X, emb, bias, w0):
    xflat = X.reshape(BATCH * FIELDS)
    w0b = jnp.broadcast_to(w0, (16,))
    tail = emb[N_TCHUNKS * TCOLS:, :].reshape(TAIL * K)

    table = pl.kernel(
        _transpose_body,
        out_type=jax.ShapeDtypeStruct((N_VOCAB * K,), jnp.float32),
        mesh=_sc_mesh(),
        scratch_types=[
            pltpu.VMEM((16, TCOLS), jnp.float32),
            pltpu.VMEM((16, TCOLS), jnp.float32),
            pltpu.VMEM((TCOLS * K,), jnp.float32),
            pltpu.VMEM((TCOLS * K,), jnp.float32),
            pltpu.SemaphoreType.DMA,
            pltpu.SemaphoreType.DMA,
            pltpu.SemaphoreType.DMA,
            pltpu.SemaphoreType.DMA,
        ],
        compiler_params=pltpu.CompilerParams(
            needs_layout_passes=False, use_tc_tiling_on_sc=True),
    )(emb.T, tail)

    return pl.kernel(
        _fm_body,
        out_type=jax.ShapeDtypeStruct((BATCH,), jnp.float32),
        mesh=_sc_mesh(),
        scratch_types=[
            pltpu.VMEM((CHUNK * FIELDS,), jnp.int32),
            pltpu.VMEM((CHUNK * FIELDS,), jnp.int32),
            pltpu.VMEM((CHUNK * FIELDS, K), jnp.float32),
            pltpu.VMEM((CHUNK * FIELDS, K), jnp.float32),
            pltpu.VMEM((CHUNK * FIELDS,), jnp.float32),
            pltpu.VMEM((CHUNK * FIELDS,), jnp.float32),
            pltpu.VMEM((CHUNK,), jnp.float32),
            pltpu.VMEM((CHUNK,), jnp.float32),
            pltpu.VMEM((16,), jnp.float32),
            pltpu.SemaphoreType.DMA,
            pltpu.SemaphoreType.DMA,
            pltpu.SemaphoreType.DMA,
            pltpu.SemaphoreType.DMA,
            pltpu.SemaphoreType.DMA,
            pltpu.SemaphoreType.DMA,
        ],
        compiler_params=pltpu.CompilerParams(
            needs_layout_passes=False, use_tc_tiling_on_sc=False),
    )(xflat, table.reshape(N_VOCAB, K), bias.T.reshape(N_VOCAB), w0b)


def kernel(X, embeddings, bias, w0):
    return _fm_call(X.astype(jnp.int32), embeddings,
                    bias.astype(jnp.float32), w0.astype(jnp.float32))
